# Initial kernel scaffold; baseline (speedup 1.0000x reference)
import jax
import jax.numpy as jnp
from jax.experimental import pallas as pl


def kernel(pos_head, pos_rel, pos_tail, neg_head, neg_rel, neg_tail, entity_emb, relation_emb):
    raise NotImplementedError("write your pallas kernel here")


# speedup vs baseline: 1.7743x; 1.7743x over previous
import functools

import jax
import jax.numpy as jnp
from jax import lax
from jax.experimental import pallas as pl
from jax.experimental.pallas import tpu as pltpu
from jax.experimental.pallas import tpu_sc as plsc

NUM_ENTITIES = 1000000
EMBED_DIM = 64
LANES = 16
CHUNK = 128
NQ = EMBED_DIM // LANES


def _rsqrt(x):
    i = lax.bitcast_convert_type(x, jnp.int32)
    i = jnp.int32(0x5F3759DF) - lax.shift_right_arithmetic(i, 1)
    y = lax.bitcast_convert_type(i, jnp.float32)
    for _ in range(3):
        y = y * (jnp.float32(1.5) - jnp.float32(0.5) * x * y * y)
    return y


def _inv_norm(rows, i, idx_scalar):
    q = [rows[i, pl.ds(k * LANES, LANES)] for k in range(NQ)]
    v = q[0] * q[0]
    for k in range(1, NQ):
        v = v + q[k] * q[k]
    ssq = jnp.sum(v)
    ssq = jnp.where(idx_scalar == NUM_ENTITIES - 1, jnp.float32(1.0), ssq)
    return _rsqrt(ssq), q


def _distance(h_rows, r_rows, t_rows, hi, ti, i):
    inv_h, hq = _inv_norm(h_rows, i, hi)
    inv_t, tq = _inv_norm(t_rows, i, ti)
    acc = None
    for k in range(NQ):
        rq = r_rows[i, pl.ds(k * LANES, LANES)]
        d = jnp.abs(hq[k] * inv_h + rq - tq[k] * inv_t)
        acc = d if acc is None else acc + d
    return jnp.sum(acc)


def kernel(pos_head, pos_rel, pos_tail, neg_head, neg_rel, neg_tail,
           entity_emb, relation_emb):
    batch = pos_head.shape[0]
    info = plsc.get_sparse_core_info()
    nc, ns = info.num_cores, info.num_subcores
    nw = nc * ns
    per_w = batch // nw
    n_chunks = per_w // CHUNK

    mesh = plsc.VectorSubcoreMesh(core_axis_name="c", subcore_axis_name="s")

    @functools.partial(
        pl.kernel,
        mesh=mesh,
        compiler_params=pltpu.CompilerParams(
            needs_layout_passes=False, use_tc_tiling_on_sc=False),
        out_type=(jax.ShapeDtypeStruct((batch,), jnp.float32),
                  jax.ShapeDtypeStruct((batch,), jnp.float32)),
        scratch_types=(
            [pltpu.VMEM((CHUNK,), jnp.int32) for _ in range(6)]
            + [pltpu.VMEM((CHUNK, EMBED_DIM), jnp.float32) for _ in range(6)]
            + [pltpu.VMEM((CHUNK,), jnp.float32) for _ in range(2)]
            + [pltpu.SemaphoreType.DMA]
        ),
    )
    def _k(ph, pr, pt, nh, nr, nt, ent, rel, pos_out, neg_out,
           ph_i, pr_i, pt_i, nh_i, nr_i, nt_i,
           hp_v, rp_v, tp_v, hn_v, rn_v, tn_v,
           po_v, no_v, sem):
        wid = lax.axis_index("s") * nc + lax.axis_index("c")

        def chunk_body(ci, carry):
            base = wid * per_w + ci * CHUNK
            pltpu.sync_copy(ph.at[pl.ds(base, CHUNK)], ph_i)
            pltpu.sync_copy(pr.at[pl.ds(base, CHUNK)], pr_i)
            pltpu.sync_copy(pt.at[pl.ds(base, CHUNK)], pt_i)
            pltpu.sync_copy(nh.at[pl.ds(base, CHUNK)], nh_i)
            pltpu.sync_copy(nr.at[pl.ds(base, CHUNK)], nr_i)
            pltpu.sync_copy(nt.at[pl.ds(base, CHUNK)], nt_i)
            copies = [
                pltpu.async_copy(ent.at[ph_i], hp_v, sem),
                pltpu.async_copy(rel.at[pr_i], rp_v, sem),
                pltpu.async_copy(ent.at[pt_i], tp_v, sem),
                pltpu.async_copy(ent.at[nh_i], hn_v, sem),
                pltpu.async_copy(rel.at[nr_i], rn_v, sem),
                pltpu.async_copy(ent.at[nt_i], tn_v, sem),
            ]
            for cp in copies:
                cp.wait()

            def group_body(g, gcarry):
                gb = g * LANES
                phv = ph_i[pl.ds(gb, LANES)]
                ptv = pt_i[pl.ds(gb, LANES)]
                nhv = nh_i[pl.ds(gb, LANES)]
                ntv = nt_i[pl.ds(gb, LANES)]
                pd = None
                nd = None
                for j in range(LANES):
                    i = gb + j
                    lane = lax.iota(jnp.int32, LANES) == j
                    dp = _distance(hp_v, rp_v, tp_v, phv[j], ptv[j], i)
                    dn = _distance(hn_v, rn_v, tn_v, nhv[j], ntv[j], i)
                    dpb = jnp.broadcast_to(dp, (LANES,))
                    dnb = jnp.broadcast_to(dn, (LANES,))
                    pd = dpb if pd is None else jnp.where(lane, dpb, pd)
                    nd = dnb if nd is None else jnp.where(lane, dnb, nd)
                po_v[pl.ds(gb, LANES)] = pd
                no_v[pl.ds(gb, LANES)] = nd
                return gcarry

            lax.fori_loop(0, CHUNK // LANES, group_body, jnp.int32(0))
            pltpu.sync_copy(po_v, pos_out.at[pl.ds(base, CHUNK)])
            pltpu.sync_copy(no_v, neg_out.at[pl.ds(base, CHUNK)])
            return carry

        lax.fori_loop(0, n_chunks, chunk_body, jnp.int32(0))

    pos, neg = _k(pos_head, pos_rel, pos_tail, neg_head, neg_rel, neg_tail,
                  entity_emb, relation_emb)
    return pos, neg


# speedup vs baseline: 2.2113x; 1.2463x over previous
import functools

import jax
import jax.numpy as jnp
from jax import lax
from jax.experimental import pallas as pl
from jax.experimental.pallas import tpu as pltpu
from jax.experimental.pallas import tpu_sc as plsc

NUM_ENTITIES = 1000000
EMBED_DIM = 64
LANES = 16
CHUNK = 128
NQ = EMBED_DIM // LANES


def _rsqrt(x):
    i = lax.bitcast_convert_type(x, jnp.int32)
    i = jnp.int32(0x5F3759DF) - lax.shift_right_arithmetic(i, 1)
    y = lax.bitcast_convert_type(i, jnp.float32)
    for _ in range(3):
        y = y * (jnp.float32(1.5) - jnp.float32(0.5) * x * y * y)
    return y


def _inv_norm(rows, i, off, idx_scalar):
    q = [rows[i, pl.ds(off + k * LANES, LANES)] for k in range(NQ)]
    v = q[0] * q[0]
    for k in range(1, NQ):
        v = v + q[k] * q[k]
    ssq = jnp.sum(v)
    ssq = jnp.where(idx_scalar == NUM_ENTITIES - 1, jnp.float32(1.0), ssq)
    return _rsqrt(ssq), q


def _half_off(idx_scalar):
    return (lax.shift_right_logical(idx_scalar, 10) & 1) * EMBED_DIM


def _pack_row(idx_vec):
    return lax.shift_left(lax.shift_right_logical(idx_vec, 11), 10) + (
        idx_vec & 1023)


def _distance(h_rows, r_rows, t_rows, hi, ri, ti, i):
    inv_h, hq = _inv_norm(h_rows, i, _half_off(hi), hi)
    inv_t, tq = _inv_norm(t_rows, i, _half_off(ti), ti)
    roff = _half_off(ri)
    acc = None
    for k in range(NQ):
        rq = r_rows[i, pl.ds(roff + k * LANES, LANES)]
        d = jnp.abs(hq[k] * inv_h + rq - tq[k] * inv_t)
        acc = d if acc is None else acc + d
    return jnp.sum(acc)


PACK_BLK = 2048


def _pack_body(x_ref, y_ref):
    t = x_ref[...].T
    y_ref[:, 0:EMBED_DIM] = t[0:PACK_BLK // 2, :]
    y_ref[:, EMBED_DIM:2 * EMBED_DIM] = t[PACK_BLK // 2:PACK_BLK, :]


def _pack_pairs(table):
    n, d = table.shape
    grid = pl.cdiv(n, PACK_BLK)
    table_t = table.T
    return pl.pallas_call(
        _pack_body,
        grid=(grid,),
        in_specs=[pl.BlockSpec((d, PACK_BLK), lambda g: (0, g))],
        out_specs=pl.BlockSpec((PACK_BLK // 2, 2 * d), lambda g: (g, 0)),
        out_shape=jax.ShapeDtypeStruct((grid * (PACK_BLK // 2), 2 * d),
                                       jnp.float32),
    )(table_t)


def kernel(pos_head, pos_rel, pos_tail, neg_head, neg_rel, neg_tail,
           entity_emb, relation_emb):
    batch = pos_head.shape[0]
    info = plsc.get_sparse_core_info()
    nc, ns = info.num_cores, info.num_subcores
    nw = nc * ns
    per_w = batch // nw
    n_chunks = per_w // CHUNK

    ent2 = _pack_pairs(entity_emb)
    rel2 = _pack_pairs(relation_emb)

    mesh = plsc.VectorSubcoreMesh(core_axis_name="c", subcore_axis_name="s")

    @functools.partial(
        pl.kernel,
        mesh=mesh,
        compiler_params=pltpu.CompilerParams(needs_layout_passes=False),
        out_type=(jax.ShapeDtypeStruct((batch,), jnp.float32),
                  jax.ShapeDtypeStruct((batch,), jnp.float32)),
        scratch_types=(
            [pltpu.VMEM((CHUNK,), jnp.int32) for _ in range(6)]
            + [pltpu.VMEM((CHUNK,), jnp.int32) for _ in range(6)]
            + [pltpu.VMEM((CHUNK, 2 * EMBED_DIM), jnp.float32)
               for _ in range(6)]
            + [pltpu.VMEM((CHUNK,), jnp.float32) for _ in range(2)]
            + [pltpu.SemaphoreType.DMA]
        ),
    )
    def _k(ph, pr, pt, nh, nr, nt, ent, rel, pos_out, neg_out,
           ph_i, pr_i, pt_i, nh_i, nr_i, nt_i,
           ph_h, pr_h, pt_h, nh_h, nr_h, nt_h,
           hp_v, rp_v, tp_v, hn_v, rn_v, tn_v,
           po_v, no_v, sem):
        wid = lax.axis_index("s") * nc + lax.axis_index("c")

        def chunk_body(ci, carry):
            base = wid * per_w + ci * CHUNK
            pltpu.sync_copy(ph.at[pl.ds(base, CHUNK)], ph_i)
            pltpu.sync_copy(pr.at[pl.ds(base, CHUNK)], pr_i)
            pltpu.sync_copy(pt.at[pl.ds(base, CHUNK)], pt_i)
            pltpu.sync_copy(nh.at[pl.ds(base, CHUNK)], nh_i)
            pltpu.sync_copy(nr.at[pl.ds(base, CHUNK)], nr_i)
            pltpu.sync_copy(nt.at[pl.ds(base, CHUNK)], nt_i)

            def halve(g, gcarry):
                gb = g * LANES
                sl = pl.ds(gb, LANES)
                ph_h[sl] = _pack_row(ph_i[sl])
                pr_h[sl] = _pack_row(pr_i[sl])
                pt_h[sl] = _pack_row(pt_i[sl])
                nh_h[sl] = _pack_row(nh_i[sl])
                nr_h[sl] = _pack_row(nr_i[sl])
                nt_h[sl] = _pack_row(nt_i[sl])
                return gcarry

            lax.fori_loop(0, CHUNK // LANES, halve, jnp.int32(0))

            copies = [
                pltpu.async_copy(ent.at[ph_h], hp_v, sem),
                pltpu.async_copy(rel.at[pr_h], rp_v, sem),
                pltpu.async_copy(ent.at[pt_h], tp_v, sem),
                pltpu.async_copy(ent.at[nh_h], hn_v, sem),
                pltpu.async_copy(rel.at[nr_h], rn_v, sem),
                pltpu.async_copy(ent.at[nt_h], tn_v, sem),
            ]
            for cp in copies:
                cp.wait()

            def group_body(g, gcarry):
                gb = g * LANES
                phv = ph_i[pl.ds(gb, LANES)]
                prv = pr_i[pl.ds(gb, LANES)]
                ptv = pt_i[pl.ds(gb, LANES)]
                nhv = nh_i[pl.ds(gb, LANES)]
                nrv = nr_i[pl.ds(gb, LANES)]
                ntv = nt_i[pl.ds(gb, LANES)]
                pd = None
                nd = None
                for j in range(LANES):
                    i = gb + j
                    lane = lax.iota(jnp.int32, LANES) == j
                    dp = _distance(hp_v, rp_v, tp_v, phv[j], prv[j], ptv[j], i)
                    dn = _distance(hn_v, rn_v, tn_v, nhv[j], nrv[j], ntv[j], i)
                    dpb = jnp.broadcast_to(dp, (LANES,))
                    dnb = jnp.broadcast_to(dn, (LANES,))
                    pd = dpb if pd is None else jnp.where(lane, dpb, pd)
                    nd = dnb if nd is None else jnp.where(lane, dnb, nd)
                po_v[pl.ds(gb, LANES)] = pd
                no_v[pl.ds(gb, LANES)] = nd
                return gcarry

            lax.fori_loop(0, CHUNK // LANES, group_body, jnp.int32(0))
            pltpu.sync_copy(po_v, pos_out.at[pl.ds(base, CHUNK)])
            pltpu.sync_copy(no_v, neg_out.at[pl.ds(base, CHUNK)])
            return carry

        lax.fori_loop(0, n_chunks, chunk_body, jnp.int32(0))

    pos, neg = _k(pos_head, pos_rel, pos_tail, neg_head, neg_rel, neg_tail,
                  ent2, rel2)
    return pos, neg


# speedup vs baseline: 2.3466x; 1.0612x over previous
import functools

import jax
import jax.numpy as jnp
from jax import lax
from jax.experimental import pallas as pl
from jax.experimental.pallas import tpu as pltpu
from jax.experimental.pallas import tpu_sc as plsc

NUM_ENTITIES = 1000000
EMBED_DIM = 64
LANES = 16
CHUNK = 128
NQ = EMBED_DIM // LANES


def _rsqrt(x):
    i = lax.bitcast_convert_type(x, jnp.int32)
    i = jnp.int32(0x5F3759DF) - lax.shift_right_arithmetic(i, 1)
    y = lax.bitcast_convert_type(i, jnp.float32)
    for _ in range(3):
        y = y * (jnp.float32(1.5) - jnp.float32(0.5) * x * y * y)
    return y


def _inv_norm(rows, i, off, idx_scalar):
    q = [rows[i, pl.ds(off + k * LANES, LANES)] for k in range(NQ)]
    v = q[0] * q[0]
    for k in range(1, NQ):
        v = v + q[k] * q[k]
    ssq = jnp.sum(v)
    ssq = jnp.where(idx_scalar == NUM_ENTITIES - 1, jnp.float32(1.0), ssq)
    return _rsqrt(ssq), q


def _half_off(idx_scalar):
    return (lax.shift_right_logical(idx_scalar, 10) & 1) * EMBED_DIM


def _pack_row(idx_vec):
    return lax.shift_left(lax.shift_right_logical(idx_vec, 11), 10) + (
        idx_vec & 1023)


def _distance(h_rows, r_rows, t_rows, hi, ri, ti, i):
    inv_h, hq = _inv_norm(h_rows, i, _half_off(hi), hi)
    inv_t, tq = _inv_norm(t_rows, i, _half_off(ti), ti)
    roff = _half_off(ri)
    acc = None
    for k in range(NQ):
        rq = r_rows[i, pl.ds(roff + k * LANES, LANES)]
        d = jnp.abs(hq[k] * inv_h + rq - tq[k] * inv_t)
        acc = d if acc is None else acc + d
    return jnp.sum(acc)


PACK_BLK = 2048


def _pack_body(x_ref, y_ref):
    x1 = x_ref[:, 0:PACK_BLK // 2]
    x2 = x_ref[:, PACK_BLK // 2:PACK_BLK]
    d = lax.broadcasted_iota(jnp.int32, (EMBED_DIM, 2 * EMBED_DIM), 0)
    k = lax.broadcasted_iota(jnp.int32, (EMBED_DIM, 2 * EMBED_DIM), 1)
    e1 = (k == d).astype(jnp.bfloat16)
    e2 = (k == d + EMBED_DIM).astype(jnp.bfloat16)
    dn = (((0,), (0,)), ((), ()))
    y_ref[...] = (
        lax.dot_general(x1.astype(jnp.bfloat16), e1, dn,
                        preferred_element_type=jnp.float32)
        + lax.dot_general(x2.astype(jnp.bfloat16), e2, dn,
                          preferred_element_type=jnp.float32))


def _pack_pairs(table):
    n, d = table.shape
    grid = pl.cdiv(n, PACK_BLK)
    table_t = table.T
    return pl.pallas_call(
        _pack_body,
        grid=(grid,),
        in_specs=[pl.BlockSpec((d, PACK_BLK), lambda g: (0, g))],
        out_specs=pl.BlockSpec((PACK_BLK // 2, 2 * d), lambda g: (g, 0)),
        out_shape=jax.ShapeDtypeStruct((grid * (PACK_BLK // 2), 2 * d),
                                       jnp.float32),
    )(table_t)


def kernel(pos_head, pos_rel, pos_tail, neg_head, neg_rel, neg_tail,
           entity_emb, relation_emb):
    batch = pos_head.shape[0]
    info = plsc.get_sparse_core_info()
    nc, ns = info.num_cores, info.num_subcores
    nw = nc * ns
    per_w = batch // nw
    n_chunks = per_w // CHUNK

    ent2 = _pack_pairs(entity_emb)
    rel2 = _pack_pairs(relation_emb)

    mesh = plsc.VectorSubcoreMesh(core_axis_name="c", subcore_axis_name="s")

    @functools.partial(
        pl.kernel,
        mesh=mesh,
        compiler_params=pltpu.CompilerParams(needs_layout_passes=False),
        out_type=(jax.ShapeDtypeStruct((batch,), jnp.float32),
                  jax.ShapeDtypeStruct((batch,), jnp.float32)),
        scratch_types=(
            [pltpu.VMEM((CHUNK,), jnp.int32) for _ in range(6)]
            + [pltpu.VMEM((CHUNK,), jnp.int32) for _ in range(6)]
            + [pltpu.VMEM((CHUNK, 2 * EMBED_DIM), jnp.float32)
               for _ in range(6)]
            + [pltpu.VMEM((CHUNK,), jnp.float32) for _ in range(2)]
            + [pltpu.SemaphoreType.DMA]
        ),
    )
    def _k(ph, pr, pt, nh, nr, nt, ent, rel, pos_out, neg_out,
           ph_i, pr_i, pt_i, nh_i, nr_i, nt_i,
           ph_h, pr_h, pt_h, nh_h, nr_h, nt_h,
           hp_v, rp_v, tp_v, hn_v, rn_v, tn_v,
           po_v, no_v, sem):
        wid = lax.axis_index("s") * nc + lax.axis_index("c")

        def chunk_body(ci, carry):
            base = wid * per_w + ci * CHUNK
            pltpu.sync_copy(ph.at[pl.ds(base, CHUNK)], ph_i)
            pltpu.sync_copy(pr.at[pl.ds(base, CHUNK)], pr_i)
            pltpu.sync_copy(pt.at[pl.ds(base, CHUNK)], pt_i)
            pltpu.sync_copy(nh.at[pl.ds(base, CHUNK)], nh_i)
            pltpu.sync_copy(nr.at[pl.ds(base, CHUNK)], nr_i)
            pltpu.sync_copy(nt.at[pl.ds(base, CHUNK)], nt_i)

            def halve(g, gcarry):
                gb = g * LANES
                sl = pl.ds(gb, LANES)
                ph_h[sl] = _pack_row(ph_i[sl])
                pr_h[sl] = _pack_row(pr_i[sl])
                pt_h[sl] = _pack_row(pt_i[sl])
                nh_h[sl] = _pack_row(nh_i[sl])
                nr_h[sl] = _pack_row(nr_i[sl])
                nt_h[sl] = _pack_row(nt_i[sl])
                return gcarry

            lax.fori_loop(0, CHUNK // LANES, halve, jnp.int32(0))

            copies = [
                pltpu.async_copy(ent.at[ph_h], hp_v, sem),
                pltpu.async_copy(rel.at[pr_h], rp_v, sem),
                pltpu.async_copy(ent.at[pt_h], tp_v, sem),
                pltpu.async_copy(ent.at[nh_h], hn_v, sem),
                pltpu.async_copy(rel.at[nr_h], rn_v, sem),
                pltpu.async_copy(ent.at[nt_h], tn_v, sem),
            ]
            for cp in copies:
                cp.wait()

            def group_body(g, gcarry):
                gb = g * LANES
                phv = ph_i[pl.ds(gb, LANES)]
                prv = pr_i[pl.ds(gb, LANES)]
                ptv = pt_i[pl.ds(gb, LANES)]
                nhv = nh_i[pl.ds(gb, LANES)]
                nrv = nr_i[pl.ds(gb, LANES)]
                ntv = nt_i[pl.ds(gb, LANES)]
                pd = None
                nd = None
                for j in range(LANES):
                    i = gb + j
                    lane = lax.iota(jnp.int32, LANES) == j
                    dp = _distance(hp_v, rp_v, tp_v, phv[j], prv[j], ptv[j], i)
                    dn = _distance(hn_v, rn_v, tn_v, nhv[j], nrv[j], ntv[j], i)
                    dpb = jnp.broadcast_to(dp, (LANES,))
                    dnb = jnp.broadcast_to(dn, (LANES,))
                    pd = dpb if pd is None else jnp.where(lane, dpb, pd)
                    nd = dnb if nd is None else jnp.where(lane, dnb, nd)
                po_v[pl.ds(gb, LANES)] = pd
                no_v[pl.ds(gb, LANES)] = nd
                return gcarry

            lax.fori_loop(0, CHUNK // LANES, group_body, jnp.int32(0))
            pltpu.sync_copy(po_v, pos_out.at[pl.ds(base, CHUNK)])
            pltpu.sync_copy(no_v, neg_out.at[pl.ds(base, CHUNK)])
            return carry

        lax.fori_loop(0, n_chunks, chunk_body, jnp.int32(0))

    pos, neg = _k(pos_head, pos_rel, pos_tail, neg_head, neg_rel, neg_tail,
                  ent2, rel2)
    return pos, neg


# speedup vs baseline: 4.3671x; 1.8611x over previous
import functools

import jax
import jax.numpy as jnp
from jax import lax
from jax.experimental import pallas as pl
from jax.experimental.pallas import tpu as pltpu
from jax.experimental.pallas import tpu_sc as plsc

NUM_ENTITIES = 1000000
EMBED_DIM = 64
LANES = 16
CHUNK = 128
NQ = EMBED_DIM // LANES


def _rsqrt(x):
    i = lax.bitcast_convert_type(x, jnp.int32)
    i = jnp.int32(0x5F3759DF) - lax.shift_right_arithmetic(i, 1)
    y = lax.bitcast_convert_type(i, jnp.float32)
    for _ in range(3):
        y = y * (jnp.float32(1.5) - jnp.float32(0.5) * x * y * y)
    return y


def _inv_norm(rows, i, off, idx_scalar):
    q = [rows[i, pl.ds(off + k * LANES, LANES)] for k in range(NQ)]
    v = q[0] * q[0]
    for k in range(1, NQ):
        v = v + q[k] * q[k]
    ssq = jnp.sum(v)
    ssq = jnp.where(idx_scalar == NUM_ENTITIES - 1, jnp.float32(1.0), ssq)
    return _rsqrt(ssq), q


def _half_off(idx_scalar):
    return (lax.shift_right_logical(idx_scalar, 14) & 1) * EMBED_DIM


def _pack_row(idx_vec):
    return lax.shift_left(lax.shift_right_logical(idx_vec, 15), 14) + (
        idx_vec & 16383)


def _distance(h_rows, r_rows, t_rows, hi, ri, ti, i):
    inv_h, hq = _inv_norm(h_rows, i, _half_off(hi), hi)
    inv_t, tq = _inv_norm(t_rows, i, _half_off(ti), ti)
    roff = _half_off(ri)
    acc = None
    for k in range(NQ):
        rq = r_rows[i, pl.ds(roff + k * LANES, LANES)]
        d = jnp.abs(hq[k] * inv_h + rq - tq[k] * inv_t)
        acc = d if acc is None else acc + d
    return jnp.sum(acc)


PACK_BLK = 32768
PACK_HALF = PACK_BLK // 2
REL_PACK_BLK = 2048


def _pack_body(blk, x_ref, y_ref):
    x1 = x_ref[:, 0:blk // 2]
    x2 = x_ref[:, blk // 2:blk]
    d = lax.broadcasted_iota(jnp.int32, (EMBED_DIM, 2 * EMBED_DIM), 0)
    k = lax.broadcasted_iota(jnp.int32, (EMBED_DIM, 2 * EMBED_DIM), 1)
    e1 = (k == d).astype(jnp.bfloat16)
    e2 = (k == d + EMBED_DIM).astype(jnp.bfloat16)
    dn = (((0,), (0,)), ((), ()))
    y_ref[...] = (
        lax.dot_general(x1.astype(jnp.bfloat16), e1, dn,
                        preferred_element_type=jnp.float32)
        + lax.dot_general(x2.astype(jnp.bfloat16), e2, dn,
                          preferred_element_type=jnp.float32))


def _pack_pairs(table, blk):
    n, d = table.shape
    grid = pl.cdiv(n, blk)
    table_t = table.T
    return pl.pallas_call(
        functools.partial(_pack_body, blk),
        grid=(grid,),
        in_specs=[pl.BlockSpec((d, blk), lambda g: (0, g))],
        out_specs=pl.BlockSpec((blk // 2, 2 * d), lambda g: (g, 0)),
        out_shape=jax.ShapeDtypeStruct((grid * (blk // 2), 2 * d),
                                       jnp.float32),
    )(table_t)


def kernel(pos_head, pos_rel, pos_tail, neg_head, neg_rel, neg_tail,
           entity_emb, relation_emb):
    batch = pos_head.shape[0]
    info = plsc.get_sparse_core_info()
    nc, ns = info.num_cores, info.num_subcores
    nw = nc * ns
    per_w = batch // nw
    n_chunks = per_w // CHUNK

    ent2 = _pack_pairs(entity_emb, PACK_BLK)
    rel2 = _pack_pairs(relation_emb, REL_PACK_BLK)

    mesh = plsc.VectorSubcoreMesh(core_axis_name="c", subcore_axis_name="s")

    @functools.partial(
        pl.kernel,
        mesh=mesh,
        compiler_params=pltpu.CompilerParams(needs_layout_passes=False),
        out_type=(jax.ShapeDtypeStruct((batch,), jnp.float32),
                  jax.ShapeDtypeStruct((batch,), jnp.float32)),
        scratch_types=(
            [pltpu.VMEM((CHUNK,), jnp.int32) for _ in range(6)]
            + [pltpu.VMEM((CHUNK,), jnp.int32) for _ in range(6)]
            + [pltpu.VMEM((CHUNK, 2 * EMBED_DIM), jnp.float32)
               for _ in range(6)]
            + [pltpu.VMEM((CHUNK,), jnp.float32) for _ in range(2)]
            + [pltpu.SemaphoreType.DMA]
        ),
    )
    def _k(ph, pr, pt, nh, nr, nt, ent, rel, pos_out, neg_out,
           ph_i, pr_i, pt_i, nh_i, nr_i, nt_i,
           ph_h, pr_h, pt_h, nh_h, nr_h, nt_h,
           hp_v, rp_v, tp_v, hn_v, rn_v, tn_v,
           po_v, no_v, sem):
        wid = lax.axis_index("s") * nc + lax.axis_index("c")

        def chunk_body(ci, carry):
            base = wid * per_w + ci * CHUNK
            pltpu.sync_copy(ph.at[pl.ds(base, CHUNK)], ph_i)
            pltpu.sync_copy(pr.at[pl.ds(base, CHUNK)], pr_i)
            pltpu.sync_copy(pt.at[pl.ds(base, CHUNK)], pt_i)
            pltpu.sync_copy(nh.at[pl.ds(base, CHUNK)], nh_i)
            pltpu.sync_copy(nr.at[pl.ds(base, CHUNK)], nr_i)
            pltpu.sync_copy(nt.at[pl.ds(base, CHUNK)], nt_i)

            def halve(g, gcarry):
                gb = g * LANES
                sl = pl.ds(gb, LANES)
                ph_h[sl] = _pack_row(ph_i[sl])
                pr_h[sl] = _pack_row(pr_i[sl])
                pt_h[sl] = _pack_row(pt_i[sl])
                nh_h[sl] = _pack_row(nh_i[sl])
                nr_h[sl] = _pack_row(nr_i[sl])
                nt_h[sl] = _pack_row(nt_i[sl])
                return gcarry

            lax.fori_loop(0, CHUNK // LANES, halve, jnp.int32(0))

            copies = [
                pltpu.async_copy(ent.at[ph_h], hp_v, sem),
                pltpu.async_copy(rel.at[pr_h], rp_v, sem),
                pltpu.async_copy(ent.at[pt_h], tp_v, sem),
                pltpu.async_copy(ent.at[nh_h], hn_v, sem),
                pltpu.async_copy(rel.at[nr_h], rn_v, sem),
                pltpu.async_copy(ent.at[nt_h], tn_v, sem),
            ]
            for cp in copies:
                cp.wait()

            def group_body(g, gcarry):
                gb = g * LANES
                phv = ph_i[pl.ds(gb, LANES)]
                prv = pr_i[pl.ds(gb, LANES)]
                ptv = pt_i[pl.ds(gb, LANES)]
                nhv = nh_i[pl.ds(gb, LANES)]
                nrv = nr_i[pl.ds(gb, LANES)]
                ntv = nt_i[pl.ds(gb, LANES)]
                pd = None
                nd = None
                for j in range(LANES):
                    i = gb + j
                    lane = lax.iota(jnp.int32, LANES) == j
                    dp = _distance(hp_v, rp_v, tp_v, phv[j], prv[j], ptv[j], i)
                    dn = _distance(hn_v, rn_v, tn_v, nhv[j], nrv[j], ntv[j], i)
                    dpb = jnp.broadcast_to(dp, (LANES,))
                    dnb = jnp.broadcast_to(dn, (LANES,))
                    pd = dpb if pd is None else jnp.where(lane, dpb, pd)
                    nd = dnb if nd is None else jnp.where(lane, dnb, nd)
                po_v[pl.ds(gb, LANES)] = pd
                no_v[pl.ds(gb, LANES)] = nd
                return gcarry

            lax.fori_loop(0, CHUNK // LANES, group_body, jnp.int32(0))
            pltpu.sync_copy(po_v, pos_out.at[pl.ds(base, CHUNK)])
            pltpu.sync_copy(no_v, neg_out.at[pl.ds(base, CHUNK)])
            return carry

        lax.fori_loop(0, n_chunks, chunk_body, jnp.int32(0))

    pos, neg = _k(pos_head, pos_rel, pos_tail, neg_head, neg_rel, neg_tail,
                  ent2, rel2)
    return pos, neg


# speedup vs baseline: 4.4666x; 1.0228x over previous
import functools

import jax
import jax.numpy as jnp
from jax import lax
from jax.experimental import pallas as pl
from jax.experimental.pallas import tpu as pltpu
from jax.experimental.pallas import tpu_sc as plsc

NUM_ENTITIES = 1000000
EMBED_DIM = 64
LANES = 16
CHUNK = 64
GROUPS = CHUNK // LANES
NQ = EMBED_DIM // LANES

PACK_BLK = 32768
PACK_HALF = PACK_BLK // 2
PACK_SHIFT = 15
HALF_SHIFT = 14
REL_PACK_BLK = 2048


def _rsqrt_vec(x):
    i = lax.bitcast_convert_type(x, jnp.int32)
    i = jnp.int32(0x5F3759DF) - (i >> 1)
    y = lax.bitcast_convert_type(i, jnp.float32)
    for _ in range(3):
        y = y * (jnp.float32(1.5) - jnp.float32(0.5) * x * y * y)
    return y


def _pack_row(idx_vec):
    return ((idx_vec >> PACK_SHIFT) << HALF_SHIFT) + (
        idx_vec & (PACK_HALF - 1))


def _half_off(idx_vec):
    return ((idx_vec >> HALF_SHIFT) & 1) << 6


def _pack_body(blk, x_ref, y_ref):
    x1 = x_ref[:, 0:blk // 2]
    x2 = x_ref[:, blk // 2:blk]
    d = lax.broadcasted_iota(jnp.int32, (EMBED_DIM, 2 * EMBED_DIM), 0)
    k = lax.broadcasted_iota(jnp.int32, (EMBED_DIM, 2 * EMBED_DIM), 1)
    e1 = (k == d).astype(jnp.bfloat16)
    e2 = (k == d + EMBED_DIM).astype(jnp.bfloat16)
    dn = (((0,), (0,)), ((), ()))
    y_ref[...] = (
        lax.dot_general(x1.astype(jnp.bfloat16), e1, dn,
                        preferred_element_type=jnp.float32)
        + lax.dot_general(x2.astype(jnp.bfloat16), e2, dn,
                          preferred_element_type=jnp.float32))


def _pack_pairs(table, blk):
    n, d = table.shape
    grid = pl.cdiv(n, blk)
    table_t = table.T
    return pl.pallas_call(
        functools.partial(_pack_body, blk),
        grid=(grid,),
        in_specs=[pl.BlockSpec((d, blk), lambda g: (0, g))],
        out_specs=pl.BlockSpec((blk // 2, 2 * d), lambda g: (g, 0)),
        out_shape=jax.ShapeDtypeStruct((grid * (blk // 2), 2 * d),
                                       jnp.float32),
    )(table_t)


def kernel(pos_head, pos_rel, pos_tail, neg_head, neg_rel, neg_tail,
           entity_emb, relation_emb):
    batch = pos_head.shape[0]
    info = plsc.get_sparse_core_info()
    nc, ns = info.num_cores, info.num_subcores
    nw = nc * ns
    per_w = batch // nw
    n_chunks = per_w // CHUNK

    ent2 = _pack_pairs(entity_emb, PACK_BLK)
    rel2 = _pack_pairs(relation_emb, REL_PACK_BLK)

    mesh = plsc.VectorSubcoreMesh(core_axis_name="c", subcore_axis_name="s")

    @functools.partial(
        pl.kernel,
        mesh=mesh,
        compiler_params=pltpu.CompilerParams(needs_layout_passes=False),
        out_type=(jax.ShapeDtypeStruct((batch,), jnp.float32),
                  jax.ShapeDtypeStruct((batch,), jnp.float32)),
        scratch_types=(
            [pltpu.VMEM((per_w,), jnp.int32) for _ in range(6)]
            + [pltpu.VMEM((n_chunks, CHUNK), jnp.int32) for _ in range(6)]
            + [pltpu.VMEM((CHUNK, 2 * EMBED_DIM), jnp.float32)
               for _ in range(12)]
            + [pltpu.VMEM((per_w,), jnp.float32) for _ in range(2)]
            + [pltpu.SemaphoreType.DMA, pltpu.SemaphoreType.DMA]
        ),
    )
    def _k(ph, pr, pt, nh, nr, nt, ent, rel, pos_out, neg_out,
           ph_i, pr_i, pt_i, nh_i, nr_i, nt_i,
           ph_h, pr_h, pt_h, nh_h, nr_h, nt_h,
           hp_a, rp_a, tp_a, hn_a, rn_a, tn_a,
           hp_b, rp_b, tp_b, hn_b, rn_b, tn_b,
           po_v, no_v, sem_a, sem_b):
        wid = lax.axis_index("s") * nc + lax.axis_index("c")
        base_w = wid * per_w
        bufs = ((hp_a, rp_a, tp_a, hn_a, rn_a, tn_a),
                (hp_b, rp_b, tp_b, hn_b, rn_b, tn_b))
        sems = (sem_a, sem_b)
        idx_refs = (ph_i, pr_i, pt_i, nh_i, nr_i, nt_i)
        row_refs = (ph_h, pr_h, pt_h, nh_h, nr_h, nt_h)

        for src, dst in zip((ph, pr, pt, nh, nr, nt), idx_refs):
            pltpu.sync_copy(src.at[pl.ds(base_w, per_w)], dst)

        def stage(g, carry):
            ci = g >> 2
            j = g & 3
            src_sl = pl.ds(g * LANES, LANES)
            dst_sl = pl.ds(j * LANES, LANES)
            for iref, rref in zip(idx_refs, row_refs):
                rref[ci, dst_sl] = _pack_row(iref[src_sl])
            return carry

        lax.fori_loop(0, per_w // LANES, stage, jnp.int32(0))

        def fire(ci, which):
            tabs = (ent, rel, ent, ent, rel, ent)
            return [
                pltpu.async_copy(tab.at[rref.at[ci]], buf, sems[which])
                for tab, rref, buf in zip(tabs, row_refs, bufs[which])
            ]

        def drain(which):
            for tab, rref, buf in zip((ent, rel, ent, ent, rel, ent),
                                      row_refs, bufs[which]):
                pltpu.make_async_copy(tab.at[rref.at[0]], buf,
                                      sems[which]).wait()

        def lane_distance(hbuf, rbuf, tbuf, hv, tv, rid):
            hoff = _half_off(hv)
            toff = _half_off(tv)
            ssq_h = None
            ssq_t = None
            for dd in range(EMBED_DIM):
                hd = plsc.load_gather(hbuf, [rid, hoff + dd])
                td = plsc.load_gather(tbuf, [rid, toff + dd])
                ssq_h = hd * hd if ssq_h is None else ssq_h + hd * hd
                ssq_t = td * td if ssq_t is None else ssq_t + td * td
            last = jnp.int32(NUM_ENTITIES - 1)
            inv_h = _rsqrt_vec(jnp.where(hv == last, jnp.float32(1.0), ssq_h))
            inv_t = _rsqrt_vec(jnp.where(tv == last, jnp.float32(1.0), ssq_t))
            acc = None
            for dd in range(EMBED_DIM):
                hd = plsc.load_gather(hbuf, [rid, hoff + dd])
                td = plsc.load_gather(tbuf, [rid, toff + dd])
                rd = plsc.load_gather(
                    rbuf, [rid, jnp.broadcast_to(jnp.int32(dd), (LANES,))])
                term = jnp.abs(hd * inv_h + rd - td * inv_t)
                acc = term if acc is None else acc + term
            return acc

        def compute_chunk(ci, which):
            hp, rp, tp, hn, rn, tn = bufs[which]

            def group(g, carry):
                gb = ci * CHUNK + g * LANES
                sl = pl.ds(gb, LANES)
                rid = lax.iota(jnp.int32, LANES) + g * LANES
                po_v[sl] = lane_distance(hp, rp, tp, ph_i[sl], pt_i[sl], rid)
                no_v[sl] = lane_distance(hn, rn, tn, nh_i[sl], nt_i[sl], rid)
                return carry

            lax.fori_loop(0, GROUPS, group, jnp.int32(0))

        fire(jnp.int32(0), 0)

        def ring(g, carry):
            ca = g * 2
            cb = ca + 1
            fire(cb, 1)
            drain(0)
            compute_chunk(ca, 0)

            @pl.when(g < n_chunks // 2 - 1)
            def _():
                fire(ca + 2, 0)

            drain(1)
            compute_chunk(cb, 1)
            return carry

        lax.fori_loop(0, n_chunks // 2, ring, jnp.int32(0))

        pltpu.sync_copy(po_v, pos_out.at[pl.ds(base_w, per_w)])
        pltpu.sync_copy(no_v, neg_out.at[pl.ds(base_w, per_w)])

    pos, neg = _k(pos_head, pos_rel, pos_tail, neg_head, neg_rel, neg_tail,
                  ent2, rel2)
    return pos, neg


# speedup vs baseline: 5.7860x; 1.2954x over previous
import functools

import jax
import jax.numpy as jnp
from jax import lax
from jax.experimental import pallas as pl
from jax.experimental.pallas import tpu as pltpu
from jax.experimental.pallas import tpu_sc as plsc

NUM_ENTITIES = 1000000
EMBED_DIM = 64
LANES = 16
CHUNK = 64
GROUPS = CHUNK // LANES
NQ = EMBED_DIM // LANES

PACK_BLK = 32768
PACK_HALF = PACK_BLK // 2
PACK_SHIFT = 15
HALF_SHIFT = 14
REL_PACK_BLK = 2048


def _rsqrt_vec(x):
    i = lax.bitcast_convert_type(x, jnp.int32)
    i = jnp.int32(0x5F3759DF) - (i >> 1)
    y = lax.bitcast_convert_type(i, jnp.float32)
    for _ in range(3):
        y = y * (jnp.float32(1.5) - jnp.float32(0.5) * x * y * y)
    return y


def _pack_row(idx_vec):
    return ((idx_vec >> PACK_SHIFT) << HALF_SHIFT) + (
        idx_vec & (PACK_HALF - 1))


def _half_off(idx_vec):
    return ((idx_vec >> HALF_SHIFT) & 1) << 6


def _pack_body(blk, x_ref, y_ref):
    x1 = x_ref[:, 0:blk // 2]
    x2 = x_ref[:, blk // 2:blk]
    d = lax.broadcasted_iota(jnp.int32, (EMBED_DIM, 2 * EMBED_DIM), 0)
    k = lax.broadcasted_iota(jnp.int32, (EMBED_DIM, 2 * EMBED_DIM), 1)
    e1 = (k == d).astype(jnp.bfloat16)
    e2 = (k == d + EMBED_DIM).astype(jnp.bfloat16)
    dn = (((0,), (0,)), ((), ()))
    y_ref[...] = (
        lax.dot_general(x1.astype(jnp.bfloat16), e1, dn,
                        preferred_element_type=jnp.float32)
        + lax.dot_general(x2.astype(jnp.bfloat16), e2, dn,
                          preferred_element_type=jnp.float32))


def _pack_pairs(table, blk):
    n, d = table.shape
    grid = pl.cdiv(n, blk)
    table_t = table.T
    return pl.pallas_call(
        functools.partial(_pack_body, blk),
        grid=(grid,),
        in_specs=[pl.BlockSpec((d, blk), lambda g: (0, g))],
        out_specs=pl.BlockSpec((blk // 2, 2 * d), lambda g: (g, 0)),
        out_shape=jax.ShapeDtypeStruct((grid * (blk // 2), 2 * d),
                                       jnp.float32),
    )(table_t)


def kernel(pos_head, pos_rel, pos_tail, neg_head, neg_rel, neg_tail,
           entity_emb, relation_emb):
    batch = pos_head.shape[0]
    info = plsc.get_sparse_core_info()
    nc, ns = info.num_cores, info.num_subcores
    nw = nc * ns
    per_w = batch // nw
    n_chunks = per_w // CHUNK

    ent2 = _pack_pairs(entity_emb, PACK_BLK)
    rel2 = _pack_pairs(relation_emb, REL_PACK_BLK)

    mesh = plsc.VectorSubcoreMesh(core_axis_name="c", subcore_axis_name="s")

    @functools.partial(
        pl.kernel,
        mesh=mesh,
        compiler_params=pltpu.CompilerParams(needs_layout_passes=False),
        out_type=(jax.ShapeDtypeStruct((batch,), jnp.float32),
                  jax.ShapeDtypeStruct((batch,), jnp.float32)),
        scratch_types=(
            [pltpu.VMEM((per_w,), jnp.int32) for _ in range(6)]
            + [pltpu.VMEM((n_chunks, CHUNK), jnp.int32) for _ in range(6)]
            + [pltpu.VMEM((CHUNK, 2 * EMBED_DIM), jnp.float32)
               for _ in range(12)]
            + [pltpu.VMEM((per_w,), jnp.float32) for _ in range(2)]
            + [pltpu.SemaphoreType.DMA, pltpu.SemaphoreType.DMA]
        ),
    )
    def _k(ph, pr, pt, nh, nr, nt, ent, rel, pos_out, neg_out,
           ph_i, pr_i, pt_i, nh_i, nr_i, nt_i,
           ph_h, pr_h, pt_h, nh_h, nr_h, nt_h,
           hp_a, rp_a, tp_a, hn_a, rn_a, tn_a,
           hp_b, rp_b, tp_b, hn_b, rn_b, tn_b,
           po_v, no_v, sem_a, sem_b):
        wid = lax.axis_index("s") * nc + lax.axis_index("c")
        base_w = wid * per_w
        bufs = ((hp_a, rp_a, tp_a, hn_a, rn_a, tn_a),
                (hp_b, rp_b, tp_b, hn_b, rn_b, tn_b))
        sems = (sem_a, sem_b)
        idx_refs = (ph_i, pr_i, pt_i, nh_i, nr_i, nt_i)
        row_refs = (ph_h, pr_h, pt_h, nh_h, nr_h, nt_h)

        for src, dst in zip((ph, pr, pt, nh, nr, nt), idx_refs):
            pltpu.sync_copy(src.at[pl.ds(base_w, per_w)], dst)

        def stage(g, carry):
            ci = g >> 2
            j = g & 3
            src_sl = pl.ds(g * LANES, LANES)
            dst_sl = pl.ds(j * LANES, LANES)
            for iref, rref in zip(idx_refs, row_refs):
                rref[ci, dst_sl] = _pack_row(iref[src_sl])
            return carry

        lax.fori_loop(0, per_w // LANES, stage, jnp.int32(0))

        def fire(ci, which):
            tabs = (ent, rel, ent, ent, rel, ent)
            return [
                pltpu.async_copy(tab.at[rref.at[ci]], buf, sems[which])
                for tab, rref, buf in zip(tabs, row_refs, bufs[which])
            ]

        def drain(which):
            for tab, rref, buf in zip((ent, rel, ent, ent, rel, ent),
                                      row_refs, bufs[which]):
                pltpu.make_async_copy(tab.at[rref.at[0]], buf,
                                      sems[which]).wait()

        def lane_distance(hbuf, rbuf, tbuf, hv, tv, rid):
            rot = lax.iota(jnp.int32, LANES)
            hoff = _half_off(hv)
            toff = _half_off(tv)
            ssq_h = None
            ssq_t = None
            for dd in range(EMBED_DIM):
                dcol = (rot + dd) & (EMBED_DIM - 1)
                hd = plsc.load_gather(hbuf, [rid, hoff + dcol])
                td = plsc.load_gather(tbuf, [rid, toff + dcol])
                ssq_h = hd * hd if ssq_h is None else ssq_h + hd * hd
                ssq_t = td * td if ssq_t is None else ssq_t + td * td
            last = jnp.int32(NUM_ENTITIES - 1)
            inv_h = _rsqrt_vec(jnp.where(hv == last, jnp.float32(1.0), ssq_h))
            inv_t = _rsqrt_vec(jnp.where(tv == last, jnp.float32(1.0), ssq_t))
            acc = None
            for dd in range(EMBED_DIM):
                dcol = (rot + dd) & (EMBED_DIM - 1)
                hd = plsc.load_gather(hbuf, [rid, hoff + dcol])
                td = plsc.load_gather(tbuf, [rid, toff + dcol])
                rd = plsc.load_gather(rbuf, [rid, dcol])
                term = jnp.abs(hd * inv_h + rd - td * inv_t)
                acc = term if acc is None else acc + term
            return acc

        def compute_chunk(ci, which):
            hp, rp, tp, hn, rn, tn = bufs[which]

            def group(g, carry):
                gb = ci * CHUNK + g * LANES
                sl = pl.ds(gb, LANES)
                rid = lax.iota(jnp.int32, LANES) + g * LANES
                po_v[sl] = lane_distance(hp, rp, tp, ph_i[sl], pt_i[sl], rid)
                no_v[sl] = lane_distance(hn, rn, tn, nh_i[sl], nt_i[sl], rid)
                return carry

            lax.fori_loop(0, GROUPS, group, jnp.int32(0))

        fire(jnp.int32(0), 0)

        def ring(g, carry):
            ca = g * 2
            cb = ca + 1
            fire(cb, 1)
            drain(0)
            compute_chunk(ca, 0)

            @pl.when(g < n_chunks // 2 - 1)
            def _():
                fire(ca + 2, 0)

            drain(1)
            compute_chunk(cb, 1)
            return carry

        lax.fori_loop(0, n_chunks // 2, ring, jnp.int32(0))

        pltpu.sync_copy(po_v, pos_out.at[pl.ds(base_w, per_w)])
        pltpu.sync_copy(no_v, neg_out.at[pl.ds(base_w, per_w)])

    pos, neg = _k(pos_head, pos_rel, pos_tail, neg_head, neg_rel, neg_tail,
                  ent2, rel2)
    return pos, neg


# speedup vs baseline: 6.0565x; 1.0468x over previous
import functools

import jax
import jax.numpy as jnp
from jax import lax
from jax.experimental import pallas as pl
from jax.experimental.pallas import tpu as pltpu
from jax.experimental.pallas import tpu_sc as plsc

NUM_ENTITIES = 1000000
EMBED_DIM = 64
LANES = 16
CHUNK = 64
GROUPS = CHUNK // LANES
NQ = EMBED_DIM // LANES

PACK_BLK = 32768
PACK_HALF = PACK_BLK // 2
PACK_SHIFT = 15
HALF_SHIFT = 14
REL_PACK_BLK = 2048


def _rsqrt_vec(x):
    i = lax.bitcast_convert_type(x, jnp.int32)
    i = jnp.int32(0x5F3759DF) - (i >> 1)
    y = lax.bitcast_convert_type(i, jnp.float32)
    for _ in range(3):
        y = y * (jnp.float32(1.5) - jnp.float32(0.5) * x * y * y)
    return y


def _pack_row(idx_vec):
    return ((idx_vec >> PACK_SHIFT) << HALF_SHIFT) + (
        idx_vec & (PACK_HALF - 1))


def _half_off(idx_vec):
    return ((idx_vec >> HALF_SHIFT) & 1) << 6


def _pack_body(blk, x_ref, y_ref):
    x1 = x_ref[:, 0:blk // 2]
    x2 = x_ref[:, blk // 2:blk]
    d = lax.broadcasted_iota(jnp.int32, (EMBED_DIM, 2 * EMBED_DIM), 0)
    k = lax.broadcasted_iota(jnp.int32, (EMBED_DIM, 2 * EMBED_DIM), 1)
    e1 = (k == d).astype(jnp.bfloat16)
    e2 = (k == d + EMBED_DIM).astype(jnp.bfloat16)
    dn = (((0,), (0,)), ((), ()))
    y_ref[...] = (
        lax.dot_general(x1.astype(jnp.bfloat16), e1, dn,
                        preferred_element_type=jnp.float32)
        + lax.dot_general(x2.astype(jnp.bfloat16), e2, dn,
                          preferred_element_type=jnp.float32))


def _pack_pairs(table, blk):
    n, d = table.shape
    grid = pl.cdiv(n, blk)
    table_t = table.T
    return pl.pallas_call(
        functools.partial(_pack_body, blk),
        grid=(grid,),
        in_specs=[pl.BlockSpec((d, blk), lambda g: (0, g))],
        out_specs=pl.BlockSpec((blk // 2, 2 * d), lambda g: (g, 0)),
        out_shape=jax.ShapeDtypeStruct((grid * (blk // 2), 2 * d),
                                       jnp.float32),
    )(table_t)


def kernel(pos_head, pos_rel, pos_tail, neg_head, neg_rel, neg_tail,
           entity_emb, relation_emb):
    batch = pos_head.shape[0]
    info = plsc.get_sparse_core_info()
    nc, ns = info.num_cores, info.num_subcores
    nw = nc * ns
    per_w = batch // nw
    n_chunks = per_w // CHUNK

    ent2 = _pack_pairs(entity_emb, PACK_BLK).reshape(-1, EMBED_DIM)
    rel2 = _pack_pairs(relation_emb, REL_PACK_BLK).reshape(-1, EMBED_DIM)

    mesh = plsc.VectorSubcoreMesh(core_axis_name="c", subcore_axis_name="s")

    @functools.partial(
        pl.kernel,
        mesh=mesh,
        compiler_params=pltpu.CompilerParams(
            needs_layout_passes=False, use_tc_tiling_on_sc=False),
        out_type=(jax.ShapeDtypeStruct((batch,), jnp.float32),
                  jax.ShapeDtypeStruct((batch,), jnp.float32)),
        scratch_types=(
            [pltpu.VMEM((per_w,), jnp.int32) for _ in range(6)]
            + [pltpu.VMEM((n_chunks, CHUNK), jnp.int32) for _ in range(6)]
            + [pltpu.VMEM((CHUNK, EMBED_DIM), jnp.float32)
               for _ in range(12)]
            + [pltpu.VMEM((per_w,), jnp.float32) for _ in range(2)]
            + [pltpu.SemaphoreType.DMA, pltpu.SemaphoreType.DMA]
        ),
    )
    def _k(ph, pr, pt, nh, nr, nt, ent, rel, pos_out, neg_out,
           ph_i, pr_i, pt_i, nh_i, nr_i, nt_i,
           ph_h, pr_h, pt_h, nh_h, nr_h, nt_h,
           hp_a, rp_a, tp_a, hn_a, rn_a, tn_a,
           hp_b, rp_b, tp_b, hn_b, rn_b, tn_b,
           po_v, no_v, sem_a, sem_b):
        wid = lax.axis_index("s") * nc + lax.axis_index("c")
        base_w = wid * per_w
        bufs = ((hp_a, rp_a, tp_a, hn_a, rn_a, tn_a),
                (hp_b, rp_b, tp_b, hn_b, rn_b, tn_b))
        sems = (sem_a, sem_b)
        idx_refs = (ph_i, pr_i, pt_i, nh_i, nr_i, nt_i)
        row_refs = (ph_h, pr_h, pt_h, nh_h, nr_h, nt_h)

        for src, dst in zip((ph, pr, pt, nh, nr, nt), idx_refs):
            pltpu.sync_copy(src.at[pl.ds(base_w, per_w)], dst)

        def stage(g, carry):
            ci = g >> 2
            j = g & 3
            src_sl = pl.ds(g * LANES, LANES)
            dst_sl = pl.ds(j * LANES, LANES)
            for iref, rref in zip(idx_refs, row_refs):
                ev = iref[src_sl]
                rref[ci, dst_sl] = (_pack_row(ev) << 1) | (
                    (ev >> HALF_SHIFT) & 1)
            return carry

        lax.fori_loop(0, per_w // LANES, stage, jnp.int32(0))

        def fire(ci, which):
            tabs = (ent, rel, ent, ent, rel, ent)
            return [
                pltpu.async_copy(tab.at[rref.at[ci]], buf, sems[which])
                for tab, rref, buf in zip(tabs, row_refs, bufs[which])
            ]

        def drain(which):
            for tab, rref, buf in zip((ent, rel, ent, ent, rel, ent),
                                      row_refs, bufs[which]):
                pltpu.make_async_copy(tab.at[rref.at[0]], buf,
                                      sems[which]).wait()

        def lane_distance(hbuf, rbuf, tbuf, hv, tv, rid):
            rot = lax.iota(jnp.int32, LANES)
            ssq_h = None
            ssq_t = None
            for dd in range(EMBED_DIM):
                dcol = (rot + dd) & (EMBED_DIM - 1)
                hd = plsc.load_gather(hbuf, [rid, dcol])
                td = plsc.load_gather(tbuf, [rid, dcol])
                ssq_h = hd * hd if ssq_h is None else ssq_h + hd * hd
                ssq_t = td * td if ssq_t is None else ssq_t + td * td
            last = jnp.int32(NUM_ENTITIES - 1)
            inv_h = _rsqrt_vec(jnp.where(hv == last, jnp.float32(1.0), ssq_h))
            inv_t = _rsqrt_vec(jnp.where(tv == last, jnp.float32(1.0), ssq_t))
            acc = None
            for dd in range(EMBED_DIM):
                dcol = (rot + dd) & (EMBED_DIM - 1)
                hd = plsc.load_gather(hbuf, [rid, dcol])
                td = plsc.load_gather(tbuf, [rid, dcol])
                rd = plsc.load_gather(rbuf, [rid, dcol])
                term = jnp.abs(hd * inv_h + rd - td * inv_t)
                acc = term if acc is None else acc + term
            return acc

        def compute_chunk(ci, which):
            hp, rp, tp, hn, rn, tn = bufs[which]

            def group(g, carry):
                gb = ci * CHUNK + g * LANES
                sl = pl.ds(gb, LANES)
                rid = lax.iota(jnp.int32, LANES) + g * LANES
                po_v[sl] = lane_distance(hp, rp, tp, ph_i[sl], pt_i[sl], rid)
                no_v[sl] = lane_distance(hn, rn, tn, nh_i[sl], nt_i[sl], rid)
                return carry

            lax.fori_loop(0, GROUPS, group, jnp.int32(0))

        fire(jnp.int32(0), 0)

        def ring(g, carry):
            ca = g * 2
            cb = ca + 1
            fire(cb, 1)
            drain(0)
            compute_chunk(ca, 0)

            @pl.when(g < n_chunks // 2 - 1)
            def _():
                fire(ca + 2, 0)

            drain(1)
            compute_chunk(cb, 1)
            return carry

        lax.fori_loop(0, n_chunks // 2, ring, jnp.int32(0))

        pltpu.sync_copy(po_v, pos_out.at[pl.ds(base_w, per_w)])
        pltpu.sync_copy(no_v, neg_out.at[pl.ds(base_w, per_w)])

    pos, neg = _k(pos_head, pos_rel, pos_tail, neg_head, neg_rel, neg_tail,
                  ent2, rel2)
    return pos, neg


# speedup vs baseline: 6.1220x; 1.0108x over previous
import functools

import jax
import jax.numpy as jnp
from jax import lax
from jax.experimental import pallas as pl
from jax.experimental.pallas import tpu as pltpu
from jax.experimental.pallas import tpu_sc as plsc

NUM_ENTITIES = 1000000
EMBED_DIM = 64
LANES = 16
CHUNK = 128
GROUPS = CHUNK // LANES
NQ = EMBED_DIM // LANES

PACK_BLK = 32768
PACK_HALF = PACK_BLK // 2
PACK_SHIFT = 15
HALF_SHIFT = 14
REL_PACK_BLK = 2048


def _rsqrt_vec(x):
    i = lax.bitcast_convert_type(x, jnp.int32)
    i = jnp.int32(0x5F3759DF) - (i >> 1)
    y = lax.bitcast_convert_type(i, jnp.float32)
    for _ in range(3):
        y = y * (jnp.float32(1.5) - jnp.float32(0.5) * x * y * y)
    return y


def _pack_row(idx_vec):
    return ((idx_vec >> PACK_SHIFT) << HALF_SHIFT) + (
        idx_vec & (PACK_HALF - 1))


def _half_off(idx_vec):
    return ((idx_vec >> HALF_SHIFT) & 1) << 6


def _pack_body(blk, x_ref, y_ref):
    x1 = x_ref[:, 0:blk // 2]
    x2 = x_ref[:, blk // 2:blk]
    d = lax.broadcasted_iota(jnp.int32, (EMBED_DIM, 2 * EMBED_DIM), 0)
    k = lax.broadcasted_iota(jnp.int32, (EMBED_DIM, 2 * EMBED_DIM), 1)
    e1 = (k == d).astype(jnp.bfloat16)
    e2 = (k == d + EMBED_DIM).astype(jnp.bfloat16)
    dn = (((0,), (0,)), ((), ()))
    y_ref[...] = (
        lax.dot_general(x1.astype(jnp.bfloat16), e1, dn,
                        preferred_element_type=jnp.float32)
        + lax.dot_general(x2.astype(jnp.bfloat16), e2, dn,
                          preferred_element_type=jnp.float32))


def _pack_pairs(table, blk):
    n, d = table.shape
    grid = pl.cdiv(n, blk)
    table_t = table.T
    return pl.pallas_call(
        functools.partial(_pack_body, blk),
        grid=(grid,),
        in_specs=[pl.BlockSpec((d, blk), lambda g: (0, g))],
        out_specs=pl.BlockSpec((blk // 2, 2 * d), lambda g: (g, 0)),
        out_shape=jax.ShapeDtypeStruct((grid * (blk // 2), 2 * d),
                                       jnp.float32),
    )(table_t)


def kernel(pos_head, pos_rel, pos_tail, neg_head, neg_rel, neg_tail,
           entity_emb, relation_emb):
    batch = pos_head.shape[0]
    info = plsc.get_sparse_core_info()
    nc, ns = info.num_cores, info.num_subcores
    nw = nc * ns
    per_w = batch // nw
    n_chunks = per_w // CHUNK

    ent2 = _pack_pairs(entity_emb, PACK_BLK).reshape(-1, EMBED_DIM)
    rel2 = _pack_pairs(relation_emb, REL_PACK_BLK).reshape(-1, EMBED_DIM)

    mesh = plsc.VectorSubcoreMesh(core_axis_name="c", subcore_axis_name="s")

    @functools.partial(
        pl.kernel,
        mesh=mesh,
        compiler_params=pltpu.CompilerParams(
            needs_layout_passes=False, use_tc_tiling_on_sc=False),
        out_type=(jax.ShapeDtypeStruct((batch,), jnp.float32),
                  jax.ShapeDtypeStruct((batch,), jnp.float32)),
        scratch_types=(
            [pltpu.VMEM((per_w,), jnp.int32) for _ in range(6)]
            + [pltpu.VMEM((n_chunks, CHUNK), jnp.int32) for _ in range(6)]
            + [pltpu.VMEM((CHUNK, EMBED_DIM), jnp.float32)
               for _ in range(12)]
            + [pltpu.VMEM((per_w,), jnp.float32) for _ in range(2)]
            + [pltpu.SemaphoreType.DMA, pltpu.SemaphoreType.DMA]
        ),
    )
    def _k(ph, pr, pt, nh, nr, nt, ent, rel, pos_out, neg_out,
           ph_i, pr_i, pt_i, nh_i, nr_i, nt_i,
           ph_h, pr_h, pt_h, nh_h, nr_h, nt_h,
           hp_a, rp_a, tp_a, hn_a, rn_a, tn_a,
           hp_b, rp_b, tp_b, hn_b, rn_b, tn_b,
           po_v, no_v, sem_a, sem_b):
        wid = lax.axis_index("s") * nc + lax.axis_index("c")
        base_w = wid * per_w
        bufs = ((hp_a, rp_a, tp_a, hn_a, rn_a, tn_a),
                (hp_b, rp_b, tp_b, hn_b, rn_b, tn_b))
        sems = (sem_a, sem_b)
        idx_refs = (ph_i, pr_i, pt_i, nh_i, nr_i, nt_i)
        row_refs = (ph_h, pr_h, pt_h, nh_h, nr_h, nt_h)

        idx_copies = [
            pltpu.async_copy(src.at[pl.ds(base_w, per_w)], dst, sem_a)
            for src, dst in zip((ph, pr, pt, nh, nr, nt), idx_refs)
        ]
        for cp in idx_copies:
            cp.wait()

        gshift = GROUPS.bit_length() - 1

        def stage(g, carry):
            ci = g >> gshift
            j = g & (GROUPS - 1)
            src_sl = pl.ds(g * LANES, LANES)
            dst_sl = pl.ds(j * LANES, LANES)
            for iref, rref in zip(idx_refs, row_refs):
                ev = iref[src_sl]
                rref[ci, dst_sl] = (_pack_row(ev) << 1) | (
                    (ev >> HALF_SHIFT) & 1)
            return carry

        lax.fori_loop(0, per_w // LANES, stage, jnp.int32(0))

        def fire(ci, which):
            tabs = (ent, rel, ent, ent, rel, ent)
            return [
                pltpu.async_copy(tab.at[rref.at[ci]], buf, sems[which])
                for tab, rref, buf in zip(tabs, row_refs, bufs[which])
            ]

        def drain(which):
            for tab, rref, buf in zip((ent, rel, ent, ent, rel, ent),
                                      row_refs, bufs[which]):
                pltpu.make_async_copy(tab.at[rref.at[0]], buf,
                                      sems[which]).wait()

        def lane_distance(hbuf, rbuf, tbuf, hv, tv, rid):
            rot = lax.iota(jnp.int32, LANES)
            ssq_h = None
            ssq_t = None
            for dd in range(EMBED_DIM):
                dcol = (rot + dd) & (EMBED_DIM - 1)
                hd = plsc.load_gather(hbuf, [rid, dcol])
                td = plsc.load_gather(tbuf, [rid, dcol])
                ssq_h = hd * hd if ssq_h is None else ssq_h + hd * hd
                ssq_t = td * td if ssq_t is None else ssq_t + td * td
            last = jnp.int32(NUM_ENTITIES - 1)
            inv_h = _rsqrt_vec(jnp.where(hv == last, jnp.float32(1.0), ssq_h))
            inv_t = _rsqrt_vec(jnp.where(tv == last, jnp.float32(1.0), ssq_t))
            acc = None
            for dd in range(EMBED_DIM):
                dcol = (rot + dd) & (EMBED_DIM - 1)
                hd = plsc.load_gather(hbuf, [rid, dcol])
                td = plsc.load_gather(tbuf, [rid, dcol])
                rd = plsc.load_gather(rbuf, [rid, dcol])
                term = jnp.abs(hd * inv_h + rd - td * inv_t)
                acc = term if acc is None else acc + term
            return acc

        def compute_chunk(ci, which):
            hp, rp, tp, hn, rn, tn = bufs[which]

            def group(g, carry):
                gb = ci * CHUNK + g * LANES
                sl = pl.ds(gb, LANES)
                rid = lax.iota(jnp.int32, LANES) + g * LANES
                po_v[sl] = lane_distance(hp, rp, tp, ph_i[sl], pt_i[sl], rid)
                no_v[sl] = lane_distance(hn, rn, tn, nh_i[sl], nt_i[sl], rid)
                return carry

            lax.fori_loop(0, GROUPS, group, jnp.int32(0))

        fire(jnp.int32(0), 0)

        def ring(g, carry):
            ca = g * 2
            cb = ca + 1
            fire(cb, 1)
            drain(0)
            compute_chunk(ca, 0)

            @pl.when(g < n_chunks // 2 - 1)
            def _():
                fire(ca + 2, 0)

            drain(1)
            compute_chunk(cb, 1)
            return carry

        lax.fori_loop(0, n_chunks // 2, ring, jnp.int32(0))

        pltpu.sync_copy(po_v, pos_out.at[pl.ds(base_w, per_w)])
        pltpu.sync_copy(no_v, neg_out.at[pl.ds(base_w, per_w)])

    pos, neg = _k(pos_head, pos_rel, pos_tail, neg_head, neg_rel, neg_tail,
                  ent2, rel2)
    return pos, neg
